# baseline (speedup 1.0000x reference)
import functools

import jax
import jax.numpy as jnp
from jax import lax
from jax.experimental import pallas as pl
from jax.experimental.pallas import tpu as pltpu
from jax.experimental.pallas import tpu_sc as plsc

_N_WORKERS = 32
_BLK = 128
_RROWS = 1000


def _repack_body(top_ref, bot_ref, o_ref):
    dim = top_ref.shape[1]
    o_ref[:, 0:dim] = top_ref[...]
    o_ref[:, dim:2 * dim] = bot_ref[...]


@functools.lru_cache(maxsize=None)
def _make_repack(n_rows: int, dim: int):
    n_pairs = n_rows // 2
    grid = n_pairs // _RROWS
    assert grid * _RROWS == n_pairs
    half = grid
    return pl.pallas_call(
        _repack_body,
        grid=(grid,),
        in_specs=[
            pl.BlockSpec((_RROWS, dim), lambda i: (i, 0)),
            pl.BlockSpec((_RROWS, dim), lambda i: (i + half, 0)),
        ],
        out_specs=pl.BlockSpec((_RROWS, 2 * dim), lambda i: (i, 0)),
        out_shape=jax.ShapeDtypeStruct((n_pairs, 2 * dim), jnp.float32),
    )


@functools.lru_cache(maxsize=None)
def _make(n_s: int, n_b: int, dim: int, n_half: int):
    total = n_s * n_b
    n_blocks = total // _BLK
    per_w = n_blocks // _N_WORKERS
    bt_per_s = n_b // _BLK
    dg = dim // 8
    assert per_w * _N_WORKERS == n_blocks
    mesh = plsc.VectorSubcoreMesh(core_axis_name="c", subcore_axis_name="s")

    @functools.partial(
        pl.kernel,
        out_type=jax.ShapeDtypeStruct((n_s, dg, bt_per_s, 8, _BLK),
                                      jnp.float32),
        mesh=mesh,
        scratch_types=[
            pltpu.VMEM((per_w * _BLK,), jnp.int32),
            pltpu.VMEM((per_w * _BLK,), jnp.int32),
            [pltpu.VMEM((_BLK, 2 * dim), jnp.float32) for _ in range(2)],
            pltpu.VMEM((dim, _BLK), jnp.float32),
            [pltpu.SemaphoreType.DMA for _ in range(2)],
            pltpu.SemaphoreType.DMA,
        ],
        compiler_params=pltpu.CompilerParams(use_tc_tiling_on_sc=False,
                                             needs_layout_passes=False),
    )
    def gather_kernel(idx_hbm, pairs_hbm, out_hbm, idx_all, pidx_all, rows,
                      trans_v, gsem, wsem):
        wid = lax.axis_index("s") * 2 + lax.axis_index("c")
        wblk = wid * per_w

        pltpu.sync_copy(idx_hbm.at[pl.ds(wblk * _BLK, per_w * _BLK)], idx_all)
        half = jnp.int32(n_half)

        def shift_body(i, carry):
            v = idx_all[pl.ds(i * 16, 16)]
            pidx_all[pl.ds(i * 16, 16)] = jnp.where(v >= half, v - half, v)
            return carry
        lax.fori_loop(0, per_w * _BLK // 16, shift_body, 0)

        def start_gather(par, j):
            pltpu.async_copy(
                pairs_hbm.at[pidx_all.at[pl.ds(j * _BLK, _BLK)]],
                rows[par],
                gsem[par],
            )

        def wait_gather(par):
            pltpu.make_async_copy(
                pairs_hbm.at[pidx_all.at[pl.ds(0, _BLK)]],
                rows[par], gsem[par]).wait()

        lane = lax.iota(jnp.int32, 16)
        diag = [(lane + j) % 16 for j in range(16)]

        def transpose_block(par, j):
            def bt_body(b0, carry):
                row_idx = lane + b0
                par_off = jnp.where(
                    idx_all[pl.ds(j * _BLK + b0, 16)] >= half,
                    jnp.int32(dim), jnp.int32(0))
                for dt in range(dim // 16):
                    for jj in range(16):
                        col_idx = diag[jj] + dt * 16 + par_off
                        vals = plsc.load_gather(rows[par],
                                                [row_idx, col_idx])
                        plsc.store_scatter(trans_v,
                                           [diag[jj] + dt * 16, row_idx],
                                           vals)
                return carry
            lax.fori_loop(0, _BLK // 16, lambda i, c: bt_body(i * 16, c), 0)

        def write_block(j):
            blk = wblk + j
            s = blk // bt_per_s
            bt = blk % bt_per_s
            copies = [
                pltpu.async_copy(trans_v.at[pl.ds(g8 * 8, 8)],
                                 out_hbm.at[s, g8, bt], wsem)
                for g8 in range(dg)
            ]
            for c in copies:
                c.wait()

        start_gather(0, 0)
        start_gather(1, 1)

        def body(i, carry):
            for par in range(2):
                j = i * 2 + par
                wait_gather(par)
                transpose_block(par, j)
                start_gather(par, j + 2)
                write_block(j)
            return carry

        lax.fori_loop(0, per_w // 2 - 1, body, 0)

        for par in range(2):
            j = per_w - 2 + par
            wait_gather(par)
            transpose_block(par, j)
            write_block(j)

    return gather_kernel


def kernel(z, table):
    b, s = z.shape
    n_rows, dim = table.shape
    pairs = _make_repack(n_rows, dim)(table, table)
    zf = z.T.reshape(b * s).astype(jnp.int32)
    out5 = _make(s, b, dim, n_rows // 2)(zf, pairs)
    return out5.transpose(2, 4, 0, 1, 3).reshape(b, s, dim)


# speedup vs baseline: 1.4449x; 1.4449x over previous
import functools

import jax
import jax.numpy as jnp
from jax import lax
from jax.experimental import pallas as pl
from jax.experimental.pallas import tpu as pltpu
from jax.experimental.pallas import tpu_sc as plsc

_N_WORKERS = 32
_BLK = 128


@functools.lru_cache(maxsize=None)
def _make(n_s: int, n_b: int, dim: int):
    total = n_s * n_b
    n_blocks = total // _BLK
    per_w = n_blocks // _N_WORKERS
    bt_per_s = n_b // _BLK
    dg = dim // 8
    assert per_w * _N_WORKERS == n_blocks
    mesh = plsc.VectorSubcoreMesh(core_axis_name="c", subcore_axis_name="s")

    @functools.partial(
        pl.kernel,
        out_type=jax.ShapeDtypeStruct((n_s, dg, bt_per_s, 8, _BLK),
                                      jnp.float32),
        mesh=mesh,
        scratch_types=[
            pltpu.VMEM((per_w * _BLK,), jnp.int32),
            [pltpu.VMEM((_BLK, dim), jnp.float32) for _ in range(2)],
            pltpu.VMEM((dim, _BLK), jnp.float32),
            [pltpu.SemaphoreType.DMA for _ in range(2)],
            pltpu.SemaphoreType.DMA,
        ],
        compiler_params=pltpu.CompilerParams(use_tc_tiling_on_sc=False, needs_layout_passes=False),
    )
    def gather_kernel(idx_hbm, table_hbm, out_hbm, idx_all, rows, trans_v,
                      gsem, wsem):
        wid = lax.axis_index("s") * 2 + lax.axis_index("c")
        wblk = wid * per_w

        pltpu.sync_copy(idx_hbm.at[pl.ds(wblk * _BLK, per_w * _BLK)], idx_all)

        def start_gather(par, j):
            pltpu.async_copy(
                table_hbm.at[idx_all.at[pl.ds(j * _BLK, _BLK)]],
                rows[par],
                gsem[par],
            )

        def wait_gather(par):
            pltpu.make_async_copy(
                table_hbm.at[idx_all.at[pl.ds(0, _BLK)]],
                rows[par], gsem[par]).wait()

        lane = lax.iota(jnp.int32, 16)
        diag = [(lane + j) % 16 for j in range(16)]

        def transpose_block(par):
            def bt_body(b0, carry):
                row_idx = lane + b0
                for dt in range(dim // 16):
                    cols = [diag[j] + dt * 16 for j in range(16)]
                    vals = [plsc.load_gather(rows[par], [row_idx, c])
                            for c in cols]
                    for c, v in zip(cols, vals):
                        plsc.store_scatter(trans_v, [c, row_idx], v)
                return carry
            lax.fori_loop(0, _BLK // 16, lambda i, c: bt_body(i * 16, c), 0)

        def write_block(j):
            blk = wblk + j
            s = blk // bt_per_s
            bt = blk % bt_per_s
            copies = [
                pltpu.async_copy(trans_v.at[pl.ds(g8 * 8, 8), pl.ds(0, _BLK)],
                                 out_hbm.at[s, g8, bt], wsem)
                for g8 in range(dg)
            ]
            for c in copies:
                c.wait()

        start_gather(0, 0)
        start_gather(1, 1)

        def body(i, carry):
            for par in range(2):
                j = i * 2 + par
                wait_gather(par)
                transpose_block(par)
                start_gather(par, j + 2)
                write_block(j)
            return carry

        lax.fori_loop(0, per_w // 2 - 1, body, 0)

        for par in range(2):
            j = per_w - 2 + par
            wait_gather(par)
            transpose_block(par)
            write_block(j)

    return gather_kernel


def kernel(z, table):
    b, s = z.shape
    dim = table.shape[1]
    zf = z.T.reshape(b * s).astype(jnp.int32)
    out5 = _make(s, b, dim)(zf, table)
    return out5.transpose(2, 4, 0, 1, 3).reshape(b, s, dim)
